# baseline (speedup 1.0000x reference)
import jax
import jax.numpy as jnp
from jax import lax
from jax.experimental import pallas as pl
from jax.experimental.pallas import tpu as pltpu
from jax.experimental.pallas import tpu_sc as plsc

B = 16384
F_SPARSE = 26
EMB = 16
VOCAB = 100000
DENSE = 13
F_PAD = 32
SLABS = 4
FPS = 8

NC = 2
NS = 16
NW = NC * NS
N_ROWS = B * F_PAD
PER_W = N_ROWS // NW
CHUNK = 128
N_CHUNKS = PER_W // CHUNK
OUT_R = N_ROWS * EMB // 128


OROWS = CHUNK * EMB // 128


def _sc_gather_body(idx_hbm, tab_hbm, out_hbm, idx_v, rows_v, o_v, gsem):
    wid = lax.axis_index("s") * NC + lax.axis_index("c")
    base = wid * PER_W
    pltpu.sync_copy(idx_hbm.at[wid], idx_v)

    def step(j, _):
        pltpu.async_copy(tab_hbm.at[idx_v.at[j]], rows_v, gsem).wait()
        def repack(d, _):
            for m in range(FPS):
                o_v[d, pl.ds(EMB * m, EMB)] = rows_v[d * FPS + m, :]
            return 0
        lax.fori_loop(0, OROWS, repack, 0)
        start = pl.multiple_of((base + j * CHUNK) * EMB // 128, OROWS)
        pltpu.sync_copy(o_v, out_hbm.at[pl.ds(start, OROWS)])
        return 0

    lax.fori_loop(0, N_CHUNKS, step, 0)


def _sc_gather(idx, tab_flat, interpret=False):
    mesh = plsc.VectorSubcoreMesh(
        core_axis_name="c", subcore_axis_name="s",
        num_cores=NC, num_subcores=NS)
    return pl.kernel(
        _sc_gather_body,
        out_type=jax.ShapeDtypeStruct((OUT_R, 128), jnp.float32),
        mesh=mesh,
        scratch_types=[
            pltpu.VMEM((N_CHUNKS, CHUNK), jnp.int32),
            pltpu.VMEM((CHUNK, EMB), jnp.float32),
            pltpu.VMEM((OROWS, 128), jnp.float32),
            pltpu.SemaphoreType.DMA,
        ],
        compiler_params=pltpu.CompilerParams(use_tc_tiling_on_sc=False),
        interpret=interpret,
    )(idx, tab_flat)


def _mlp_body(g_ref, xd_ref, w1p_ref, w1d_ref, b1_ref, w2_ref, b2_ref,
              w3_ref, b3_ref, ww_ref, bw_ref, mix_ref,
              logit_ref, prob_ref):
    xd = xd_ref[...]
    h = jnp.dot(xd, w1d_ref[...], preferred_element_type=jnp.float32)
    for r in range(SLABS):
        h += jnp.dot(g_ref[r], w1p_ref[128 * r:128 * (r + 1), :],
                     preferred_element_type=jnp.float32)
    h = jnp.maximum(h + b1_ref[...], 0.0)
    h = jnp.maximum(
        jnp.dot(h, w2_ref[...], preferred_element_type=jnp.float32)
        + b2_ref[...], 0.0)
    deep = jnp.dot(h, w3_ref[...], preferred_element_type=jnp.float32) + b3_ref[...]
    wide = jnp.dot(xd, ww_ref[...], preferred_element_type=jnp.float32) + bw_ref[...]
    e = jnp.exp(mix_ref[...] - jnp.max(mix_ref[...]))
    w = e / jnp.sum(e)
    logit = wide * w[0:1, 0:1] + deep * w[0:1, 1:2]
    logit_ref[...] = logit
    prob_ref[...] = 1.0 / (1.0 + jnp.exp(-logit))


def _mlp(g, xd, w1p, w1d, b1, w2, b2, w3, b3, ww, bw, mix, interpret=False):
    BB = 2048
    grid = (B // BB,)
    const = lambda shape: pl.BlockSpec(shape, lambda i: tuple(0 for _ in shape))
    return pl.pallas_call(
        _mlp_body,
        grid=grid,
        in_specs=[
            pl.BlockSpec((SLABS, BB, 128), lambda i: (0, i, 0)),
            pl.BlockSpec((BB, DENSE), lambda i: (i, 0)),
            const((SLABS * 128, 64)),
            const((DENSE, 64)),
            const((1, 64)),
            const((64, 32)),
            const((1, 32)),
            const((32, 1)),
            const((1, 1)),
            const((DENSE, 1)),
            const((1, 1)),
            const((1, 2)),
        ],
        out_specs=[
            pl.BlockSpec((BB, 1), lambda i: (i, 0)),
            pl.BlockSpec((BB, 1), lambda i: (i, 0)),
        ],
        out_shape=[
            jax.ShapeDtypeStruct((B, 1), jnp.float32),
            jax.ShapeDtypeStruct((B, 1), jnp.float32),
        ],
        interpret=interpret,
    )(g, xd, w1p, w1d, b1, w2, b2, w3, b3, ww, bw, mix)


@jax.jit
def kernel(x_sparse, x_dense, tables, W_wide, b_wide, W1, b1, W2, b2, W3, b3, mix):
    tab_flat = tables.reshape(F_SPARSE * VOCAB, EMB)
    offs = (jnp.arange(F_SPARSE, dtype=jnp.int32) * VOCAB)[None, :]
    idx_pad = jnp.concatenate(
        [x_sparse.astype(jnp.int32) + offs,
         jnp.zeros((B, F_PAD - F_SPARSE), jnp.int32)], axis=1)
    idx_sm = idx_pad.reshape(B, SLABS, FPS).transpose(1, 0, 2)
    idx = idx_sm.reshape(NW, N_CHUNKS, CHUNK)
    g = _sc_gather(idx, tab_flat).reshape(SLABS, B, 128)
    w1p = jnp.concatenate(
        [W1[:F_SPARSE * EMB], jnp.zeros((SLABS * 128 - F_SPARSE * EMB, 64),
                                        jnp.float32)], axis=0)
    logit, prob = _mlp(
        g, x_dense,
        w1p, W1[F_SPARSE * EMB:], b1.reshape(1, 64),
        W2, b2.reshape(1, 32), W3, b3.reshape(1, 1),
        W_wide, b_wide.reshape(1, 1), mix.reshape(1, 2))
    return (logit, prob)
